# baseline (speedup 1.0000x reference)
import functools

import jax
import jax.numpy as jnp
from jax import lax
from jax.experimental import pallas as pl
from jax.experimental.pallas import tpu as pltpu
from jax.experimental.pallas import tpu_sc as plsc

DIM = 128
ROWS = 1000
BATCH = 16384
NC, NS = 1, 16
NW = NC * NS
B_PER_W = BATCH // NW
CHUNK = 128
NCHUNK = B_PER_W // CHUNK
NBUF = 4
STAGE_WORKERS = 5
STAGE_ROWS = ROWS // STAGE_WORKERS

_mesh = plsc.VectorSubcoreMesh(
    core_axis_name="c", subcore_axis_name="s", num_cores=NC
)


@functools.partial(
    pl.kernel,
    mesh=_mesh,
    out_type=jax.ShapeDtypeStruct((BATCH, DIM), jnp.float32),
    scratch_types=[
        pltpu.VMEM_SHARED((ROWS, DIM), jnp.float32),
        pltpu.VMEM((NCHUNK, CHUNK), jnp.int32),
        pltpu.VMEM((NBUF, CHUNK, DIM), jnp.float32),
    ]
    + [pltpu.SemaphoreType.DMA] * NBUF
    + [pltpu.SemaphoreType.DMA] * NBUF,
)
def _gather_kernel(idx_hbm, table_hbm, out_hbm, table_spm, idx_v, rows_v, *sems):
    gsems, ssems = sems[:NBUF], sems[NBUF:]
    sid = lax.axis_index("s")
    wid = sid * NC + lax.axis_index("c")
    base = wid * B_PER_W

    @pl.when(sid < STAGE_WORKERS)
    def _stage():
        off = pl.multiple_of(sid * STAGE_ROWS, 8)
        pltpu.sync_copy(
            table_hbm.at[pl.ds(off, STAGE_ROWS)],
            table_spm.at[pl.ds(off, STAGE_ROWS)],
        )

    pltpu.sync_copy(idx_hbm.at[pl.ds(wid * NCHUNK, NCHUNK)], idx_v)
    plsc.subcore_barrier()

    def gather(j, b):
        return pltpu.async_copy(
            table_spm.at[idx_v.at[j]], rows_v.at[b], gsems[b]
        )

    gh = [gather(j, j) for j in range(NBUF)]
    sh = [None] * NBUF
    for j in range(NCHUNK):
        b = j % NBUF
        gh[b].wait()
        sh[b] = pltpu.async_copy(
            rows_v.at[b], out_hbm.at[pl.ds(base + j * CHUNK, CHUNK)], ssems[b]
        )
        nxt = j + NBUF
        if nxt < NCHUNK:
            sh[b].wait()
            gh[b] = gather(nxt, b)
            sh[b] = None
    for b in range(NBUF):
        if sh[b] is not None:
            sh[b].wait()


def kernel(timesteps, pe):
    idx2d = timesteps.astype(jnp.int32).reshape(NW * NCHUNK, CHUNK)
    return _gather_kernel(idx2d, pe)


# speedup vs baseline: 1.0573x; 1.0573x over previous
import functools

import jax
import jax.numpy as jnp
from jax import lax
from jax.experimental import pallas as pl
from jax.experimental.pallas import tpu as pltpu
from jax.experimental.pallas import tpu_sc as plsc

DIM = 128
ROWS = 1000
BATCH = 16384
NC, NS = 2, 16
NW = NC * NS
B_PER_W = BATCH // NW
CHUNK = 128
NCHUNK = B_PER_W // CHUNK
NBUF = 4
STAGE_ROWS = 64
STAGE_TAIL = ROWS - 15 * STAGE_ROWS

_mesh = plsc.VectorSubcoreMesh(
    core_axis_name="c", subcore_axis_name="s", num_cores=NC
)


@functools.partial(
    pl.kernel,
    mesh=_mesh,
    out_type=jax.ShapeDtypeStruct((BATCH, DIM), jnp.float32),
    scratch_types=[
        pltpu.VMEM_SHARED((ROWS, DIM), jnp.float32),
        pltpu.VMEM((NCHUNK, CHUNK), jnp.int32),
        pltpu.VMEM((NBUF, CHUNK, DIM), jnp.float32),
    ]
    + [pltpu.SemaphoreType.DMA] * NBUF
    + [pltpu.SemaphoreType.DMA] * NBUF,
)
def _gather_kernel(idx_hbm, table_hbm, out_hbm, table_spm, idx_v, rows_v, *sems):
    gsems, ssems = sems[:NBUF], sems[NBUF:]
    sid = lax.axis_index("s")
    wid = sid * NC + lax.axis_index("c")
    base = wid * B_PER_W

    @pl.when(sid < 15)
    def _stage():
        off = pl.multiple_of(sid * STAGE_ROWS, 8)
        pltpu.sync_copy(
            table_hbm.at[pl.ds(off, STAGE_ROWS)],
            table_spm.at[pl.ds(off, STAGE_ROWS)],
        )

    @pl.when(sid == 15)
    def _stage_tail():
        off = pl.multiple_of(15 * STAGE_ROWS, 8)
        pltpu.sync_copy(
            table_hbm.at[pl.ds(off, STAGE_TAIL)],
            table_spm.at[pl.ds(off, STAGE_TAIL)],
        )

    pltpu.sync_copy(idx_hbm.at[pl.ds(wid * NCHUNK, NCHUNK)], idx_v)
    plsc.subcore_barrier()

    def gather(j, b):
        return pltpu.async_copy(
            table_spm.at[idx_v.at[j]], rows_v.at[b], gsems[b]
        )

    gh = [gather(j, j) for j in range(NBUF)]
    sh = [None] * NBUF
    for j in range(NCHUNK):
        b = j % NBUF
        gh[b].wait()
        sh[b] = pltpu.async_copy(
            rows_v.at[b], out_hbm.at[pl.ds(base + j * CHUNK, CHUNK)], ssems[b]
        )
        nxt = j + NBUF
        if nxt < NCHUNK:
            sh[b].wait()
            gh[b] = gather(nxt, b)
            sh[b] = None
    for b in range(NBUF):
        if sh[b] is not None:
            sh[b].wait()


def kernel(timesteps, pe):
    idx2d = timesteps.astype(jnp.int32).reshape(NW * NCHUNK, CHUNK)
    return _gather_kernel(idx2d, pe)


# speedup vs baseline: 1.0588x; 1.0015x over previous
import functools

import jax
import jax.numpy as jnp
from jax import lax
from jax.experimental import pallas as pl
from jax.experimental.pallas import tpu as pltpu
from jax.experimental.pallas import tpu_sc as plsc

DIM = 128
ROWS = 1000
BATCH = 16384
NC, NS = 2, 16
NW = NC * NS
B_PER_W = BATCH // NW
CHUNK = 64
NCHUNK = B_PER_W // CHUNK
NBUF = 4
STAGE_ROWS = 64
STAGE_TAIL = ROWS - 15 * STAGE_ROWS

_mesh = plsc.VectorSubcoreMesh(
    core_axis_name="c", subcore_axis_name="s", num_cores=NC
)


@functools.partial(
    pl.kernel,
    mesh=_mesh,
    out_type=jax.ShapeDtypeStruct((BATCH, DIM), jnp.float32),
    scratch_types=[
        pltpu.VMEM_SHARED((ROWS, DIM), jnp.float32),
        pltpu.VMEM((NCHUNK, CHUNK), jnp.int32),
        pltpu.VMEM((NBUF, CHUNK, DIM), jnp.float32),
    ]
    + [pltpu.SemaphoreType.DMA] * NBUF
    + [pltpu.SemaphoreType.DMA] * NBUF,
)
def _gather_kernel(idx_hbm, table_hbm, out_hbm, table_spm, idx_v, rows_v, *sems):
    gsems, ssems = sems[:NBUF], sems[NBUF:]
    sid = lax.axis_index("s")
    wid = sid * NC + lax.axis_index("c")
    base = wid * B_PER_W

    @pl.when(sid < 15)
    def _stage():
        off = pl.multiple_of(sid * STAGE_ROWS, 8)
        pltpu.sync_copy(
            table_hbm.at[pl.ds(off, STAGE_ROWS)],
            table_spm.at[pl.ds(off, STAGE_ROWS)],
        )

    @pl.when(sid == 15)
    def _stage_tail():
        off = pl.multiple_of(15 * STAGE_ROWS, 8)
        pltpu.sync_copy(
            table_hbm.at[pl.ds(off, STAGE_TAIL)],
            table_spm.at[pl.ds(off, STAGE_TAIL)],
        )

    pltpu.sync_copy(idx_hbm.at[pl.ds(wid * NCHUNK, NCHUNK)], idx_v)
    plsc.subcore_barrier()

    def gather(j, b):
        return pltpu.async_copy(
            table_spm.at[idx_v.at[j]], rows_v.at[b], gsems[b]
        )

    gh = [gather(j, j % NBUF) for j in range(min(NBUF, NCHUNK))]
    sh = [None] * NBUF
    for j in range(NCHUNK):
        b = j % NBUF
        gh[b].wait()
        sh[b] = pltpu.async_copy(
            rows_v.at[b], out_hbm.at[pl.ds(base + j * CHUNK, CHUNK)], ssems[b]
        )
        prev, nxt = j - 1, j - 1 + NBUF
        if prev >= 0 and nxt < NCHUNK:
            pb = prev % NBUF
            sh[pb].wait()
            gh[pb] = gather(nxt, pb)
            sh[pb] = None
    for b in range(NBUF):
        if sh[b] is not None:
            sh[b].wait()


def kernel(timesteps, pe):
    idx2d = timesteps.astype(jnp.int32).reshape(NW * NCHUNK, CHUNK)
    return _gather_kernel(idx2d, pe)
